# baseline (speedup 1.0000x reference)
import functools

import jax
import jax.numpy as jnp
from jax import lax
from jax.experimental import pallas as pl
from jax.experimental.pallas import tpu as pltpu
from jax.experimental.pallas import tpu_sc as plsc

EMBED_DIM = 32
N_CODES = 1000000
CB = 16384
RB = CB // 4
NUM_WORKERS = 32
B = 16384
S = 50
BW = B // NUM_WORKERS
IPW = BW * S

MESH = plsc.VectorSubcoreMesh(core_axis_name="c", subcore_axis_name="s")


def _tc_pack(emb_t):
    def body(x_ref, o_ref):
        y = jnp.transpose(x_ref[...], (1, 0))
        t = jnp.transpose(jnp.reshape(y, (RB, 4, 32)), (1, 0, 2))
        for u in range(4):
            o_ref[:, 32 * u:32 * (u + 1)] = t[u]

    return pl.pallas_call(
        body,
        grid=((N_CODES + CB - 1) // CB,),
        in_specs=[pl.BlockSpec((32, CB), lambda i: (0, i))],
        out_specs=pl.BlockSpec((RB, 128), lambda i: (i, 0)),
        out_shape=jax.ShapeDtypeStruct((N_CODES // 4, 128), jnp.float32),
    )(emb_t)


@functools.partial(
    pl.kernel,
    mesh=MESH,
    out_type=jax.ShapeDtypeStruct((S, 4, 128, 8, 128), jnp.float32),
    scratch_types=[
        pltpu.VMEM((IPW,), jnp.int32),
        pltpu.VMEM((S, BW), jnp.int32),
        [pltpu.VMEM((BW, EMBED_DIM), jnp.float32) for _ in range(2)],
        [pltpu.VMEM((EMBED_DIM, 513), jnp.float32) for _ in range(2)],
        [pltpu.SemaphoreType.DMA for _ in range(2)],
        [pltpu.SemaphoreType.DMA for _ in range(2)],
    ],
    compiler_params=pltpu.CompilerParams(
        use_tc_tiling_on_sc=False, needs_layout_passes=False
    ),
)
def _sc_gather(table_h, idx_h, out5, idx_v, sidx_v, rows, trows, gsems, wsems):
    wid = lax.axis_index("s") * 2 + lax.axis_index("c")
    pltpu.sync_copy(idx_h.at[pl.ds(wid * IPW, IPW)], idx_v)

    lane = lax.iota(jnp.int32, 16)
    base = lane * S

    def reorder(s, carry):
        for k in range(BW // 16):
            pos = base + (k * 16 * S + s)
            sidx_v[s, pl.ds(k * 16, 16)] = plsc.load_gather(idx_v, [pos])
        return carry

    lax.fori_loop(0, S, reorder, 0)

    def unit(g, b, first):
        pltpu.make_async_copy(table_h.at[pl.ds(0, BW)], rows[b], gsems[b]).wait()
        if not first:
            pltpu.make_async_copy(
                table_h.at[pl.ds(0, BW)], rows[b], wsems[b]
            ).wait()

        def tb(k, carry):
            bb0 = k * 16
            for kk in range(16):
                bb = bb0 + kk
                cidx = jnp.full((16,), bb, jnp.int32)
                for h in range(2):
                    plsc.store_scatter(
                        trows[b], [lane + h * 16, cidx],
                        rows[b][bb, pl.ds(h * 16, 16)],
                    )
            return carry

        lax.fori_loop(0, BW // 16, tb, 0)
        nxt = jnp.minimum(g + 2, S - 1)
        pltpu.async_copy(table_h.at[sidx_v.at[nxt]], rows[b], gsems[b])
        for dt in range(4):
            for bt in range(4):
                pltpu.async_copy(
                    trows[b].at[pl.ds(8 * dt, 8), pl.ds(128 * bt, 128)],
                    out5.at[g, dt, 4 * wid + bt],
                    wsems[b],
                )

    for b in range(2):
        pltpu.async_copy(table_h.at[sidx_v.at[b]], rows[b], gsems[b])
    unit(jnp.int32(0), 0, True)
    unit(jnp.int32(1), 1, True)

    def step(t, carry):
        for b in range(2):
            unit(2 * t + b, b, False)
        return carry

    lax.fori_loop(1, S // 2, step, 0)
    for b in range(2):
        pltpu.make_async_copy(table_h.at[pl.ds(0, BW)], rows[b], wsems[b]).wait()
        pltpu.make_async_copy(table_h.at[pl.ds(0, BW)], rows[b], gsems[b]).wait()


def kernel(encodings, embeddings):
    table_rm = jnp.reshape(_tc_pack(embeddings.T), (N_CODES, EMBED_DIM))
    idx = encodings.reshape(-1).astype(jnp.int32)
    out5 = _sc_gather(table_rm, idx)
    x = jnp.transpose(out5, (2, 4, 0, 1, 3))
    return jnp.reshape(x, (B, S, EMBED_DIM))


# speedup vs baseline: 1.3291x; 1.3291x over previous
import functools

import jax
import jax.numpy as jnp
from jax import lax
from jax.experimental import pallas as pl
from jax.experimental.pallas import tpu as pltpu
from jax.experimental.pallas import tpu_sc as plsc

EMBED_DIM = 32
N_CODES = 1000000
CB = 16384
RB = CB // 4
NUM_WORKERS = 32
B = 16384
S = 50
BW = B // NUM_WORKERS
IPW = BW * S

MESH = plsc.VectorSubcoreMesh(core_axis_name="c", subcore_axis_name="s")


def _tc_pack(emb_t):
    def body(x_ref, o_ref):
        y = jnp.transpose(x_ref[...], (1, 0))
        t = jnp.transpose(jnp.reshape(y, (RB, 4, 32)), (1, 0, 2))
        for u in range(4):
            o_ref[:, 32 * u:32 * (u + 1)] = t[u]

    return pl.pallas_call(
        body,
        grid=((N_CODES + CB - 1) // CB,),
        in_specs=[pl.BlockSpec((32, CB), lambda i: (0, i))],
        out_specs=pl.BlockSpec((RB, 128), lambda i: (i, 0)),
        out_shape=jax.ShapeDtypeStruct((N_CODES // 4, 128), jnp.float32),
    )(emb_t)


@functools.partial(
    pl.kernel,
    mesh=MESH,
    out_type=jax.ShapeDtypeStruct((S, 4, 128, 8, 128), jnp.float32),
    scratch_types=[
        pltpu.VMEM((IPW,), jnp.int32),
        pltpu.VMEM((S, BW), jnp.int32),
        [pltpu.VMEM((BW, EMBED_DIM), jnp.float32) for _ in range(2)],
        [pltpu.VMEM((EMBED_DIM, 513), jnp.float32) for _ in range(2)],
        [pltpu.SemaphoreType.DMA for _ in range(2)],
        [pltpu.SemaphoreType.DMA for _ in range(2)],
    ],
    compiler_params=pltpu.CompilerParams(
        use_tc_tiling_on_sc=False, needs_layout_passes=False
    ),
)
def _sc_gather(table_h, idx_h, out5, idx_v, sidx_v, rows, trows, gsems, wsems):
    wid = lax.axis_index("s") * 2 + lax.axis_index("c")
    pltpu.sync_copy(idx_h.at[pl.ds(wid * IPW, IPW)], idx_v)

    lane = lax.iota(jnp.int32, 16)
    base = lane * S

    def reorder(s, carry):
        for k in range(BW // 16):
            pos = base + (k * 16 * S + s)
            sidx_v[s, pl.ds(k * 16, 16)] = plsc.load_gather(idx_v, [pos])
        return carry

    lax.fori_loop(0, S, reorder, 0)

    def unit(g, b, first):
        pltpu.make_async_copy(table_h.at[pl.ds(0, BW)], rows[b], gsems[b]).wait()
        if not first:
            pltpu.make_async_copy(
                table_h.at[pl.ds(0, BW)], rows[b], wsems[b]
            ).wait()

        @plsc.parallel_loop(0, BW // 16, unroll=2)
        def tb(k):
            bb0 = k * 16
            for kk in range(16):
                bb = bb0 + kk
                cidx = jnp.full((16,), bb, jnp.int32)
                for h in range(2):
                    plsc.store_scatter(
                        trows[b], [lane + h * 16, cidx],
                        rows[b][bb, pl.ds(h * 16, 16)],
                    )
        nxt = jnp.minimum(g + 2, S - 1)
        pltpu.async_copy(table_h.at[sidx_v.at[nxt]], rows[b], gsems[b])
        for dt in range(4):
            for bt in range(4):
                pltpu.async_copy(
                    trows[b].at[pl.ds(8 * dt, 8), pl.ds(128 * bt, 128)],
                    out5.at[g, dt, 4 * wid + bt],
                    wsems[b],
                )

    for b in range(2):
        pltpu.async_copy(table_h.at[sidx_v.at[b]], rows[b], gsems[b])
    unit(jnp.int32(0), 0, True)
    unit(jnp.int32(1), 1, True)

    def step(t, carry):
        for b in range(2):
            unit(2 * t + b, b, False)
        return carry

    lax.fori_loop(1, S // 2, step, 0)
    for b in range(2):
        pltpu.make_async_copy(table_h.at[pl.ds(0, BW)], rows[b], wsems[b]).wait()
        pltpu.make_async_copy(table_h.at[pl.ds(0, BW)], rows[b], gsems[b]).wait()


def kernel(encodings, embeddings):
    table_rm = jnp.reshape(_tc_pack(embeddings.T), (N_CODES, EMBED_DIM))
    idx = encodings.reshape(-1).astype(jnp.int32)
    out5 = _sc_gather(table_rm, idx)
    x = jnp.transpose(out5, (2, 4, 0, 1, 3))
    return jnp.reshape(x, (B, S, EMBED_DIM))


# speedup vs baseline: 1.3485x; 1.0145x over previous
import functools

import jax
import jax.numpy as jnp
from jax import lax
from jax.experimental import pallas as pl
from jax.experimental.pallas import tpu as pltpu
from jax.experimental.pallas import tpu_sc as plsc

EMBED_DIM = 32
N_CODES = 1000000
CB = 16384
RB = CB // 4
NUM_WORKERS = 32
B = 16384
S = 50
BW = B // NUM_WORKERS
IPW = BW * S

MESH = plsc.VectorSubcoreMesh(core_axis_name="c", subcore_axis_name="s")


def _tc_pack(emb_t):
    def body(x_ref, o_ref):
        y = jnp.transpose(x_ref[...], (1, 0))
        t = jnp.transpose(jnp.reshape(y, (RB, 4, 32)), (1, 0, 2))
        for u in range(4):
            o_ref[:, 32 * u:32 * (u + 1)] = t[u]

    return pl.pallas_call(
        body,
        grid=((N_CODES + CB - 1) // CB,),
        in_specs=[pl.BlockSpec((32, CB), lambda i: (0, i))],
        out_specs=pl.BlockSpec((RB, 128), lambda i: (i, 0)),
        out_shape=jax.ShapeDtypeStruct((N_CODES // 4, 128), jnp.float32),
    )(emb_t)


@functools.partial(
    pl.kernel,
    mesh=MESH,
    out_type=jax.ShapeDtypeStruct((S, 4, 128, 8, 128), jnp.float32),
    scratch_types=[
        pltpu.VMEM((IPW,), jnp.int32),
        pltpu.VMEM((S, BW), jnp.int32),
        [pltpu.VMEM((BW, EMBED_DIM), jnp.float32) for _ in range(2)],
        [pltpu.VMEM((EMBED_DIM, 513), jnp.float32) for _ in range(2)],
        [pltpu.SemaphoreType.DMA for _ in range(2)],
        [pltpu.SemaphoreType.DMA for _ in range(2)],
    ],
    compiler_params=pltpu.CompilerParams(
        use_tc_tiling_on_sc=False, needs_layout_passes=False
    ),
)
def _sc_gather(table_h, idx_h, out5, idx_v, sidx_v, rows, trows, gsems, wsems):
    wid = lax.axis_index("s") * 2 + lax.axis_index("c")
    pltpu.sync_copy(idx_h.at[pl.ds(wid * IPW, IPW)], idx_v)

    lane = lax.iota(jnp.int32, 16)
    base = lane * S

    @plsc.parallel_loop(0, S, unroll=2)
    def reorder(s):
        for k in range(BW // 16):
            pos = base + (k * 16 * S + s)
            sidx_v[s, pl.ds(k * 16, 16)] = plsc.load_gather(idx_v, [pos])

    def unit(g, b, first):
        pltpu.make_async_copy(table_h.at[pl.ds(0, BW)], rows[b], gsems[b]).wait()
        if not first:
            pltpu.make_async_copy(
                table_h.at[pl.ds(0, BW)], rows[b], wsems[b]
            ).wait()

        @plsc.parallel_loop(0, BW // 16, unroll=2)
        def tb(k):
            bb0 = k * 16
            for kk in range(16):
                bb = bb0 + kk
                cidx = jnp.full((16,), bb, jnp.int32)
                for h in range(2):
                    plsc.store_scatter(
                        trows[b], [lane + h * 16, cidx],
                        rows[b][bb, pl.ds(h * 16, 16)],
                    )
        nxt = jnp.minimum(g + 2, S - 1)
        pltpu.async_copy(table_h.at[sidx_v.at[nxt]], rows[b], gsems[b])
        for dt in range(4):
            for bt in range(4):
                pltpu.async_copy(
                    trows[b].at[pl.ds(8 * dt, 8), pl.ds(128 * bt, 128)],
                    out5.at[g, dt, 4 * wid + bt],
                    wsems[b],
                )

    for b in range(2):
        pltpu.async_copy(table_h.at[sidx_v.at[b]], rows[b], gsems[b])
    unit(jnp.int32(0), 0, True)
    unit(jnp.int32(1), 1, True)

    def step(t, carry):
        for b in range(2):
            unit(2 * t + b, b, False)
        return carry

    lax.fori_loop(1, S // 2, step, 0)
    for b in range(2):
        pltpu.make_async_copy(table_h.at[pl.ds(0, BW)], rows[b], wsems[b]).wait()
        pltpu.make_async_copy(table_h.at[pl.ds(0, BW)], rows[b], gsems[b]).wait()


def kernel(encodings, embeddings):
    table_rm = jnp.reshape(_tc_pack(embeddings.T), (N_CODES, EMBED_DIM))
    idx = encodings.reshape(-1).astype(jnp.int32)
    out5 = _sc_gather(table_rm, idx)
    x = jnp.transpose(out5, (2, 4, 0, 1, 3))
    return jnp.reshape(x, (B, S, EMBED_DIM))
